# baseline (speedup 1.0000x reference)
import functools

import jax
import jax.numpy as jnp
from jax import lax
from jax.experimental import pallas as pl
from jax.experimental.pallas import tpu as pltpu
from jax.experimental.pallas import tpu_sc as plsc

E = 8
TOPK = 2
H = 768
I = 1536
T = 2048
S = T * TOPK
BT = 128
P = 5120
NBLK = P // BT
NBPAD = 128
NW = 32


def _router_body(x_ref, gw_ref, pos_ref, wtk_ref, emap_ref, evalid_ref,
                 xcopy_ref):
    x = x_ref[...]
    xcopy_ref[...] = x
    gw = gw_ref[...]
    logits = lax.dot_general(x, gw, (((1,), (1,)), ((), ())),
                             preferred_element_type=jnp.float32)
    m = jnp.max(logits, axis=1, keepdims=True)
    p = jnp.exp(logits - m)
    p = p / jnp.sum(p, axis=1, keepdims=True)

    lane = lax.broadcasted_iota(jnp.int32, (T, E), 1)
    m1 = jnp.max(p, axis=1, keepdims=True)
    i1 = jnp.min(jnp.where(p == m1, lane, E), axis=1, keepdims=True)
    p2 = jnp.where(lane == i1, -jnp.inf, p)
    m2 = jnp.max(p2, axis=1, keepdims=True)
    i2 = jnp.min(jnp.where(p2 == m2, lane, E), axis=1, keepdims=True)
    ssum = m1 + m2
    wtk_ref[...] = jnp.concatenate([m1 / ssum, m2 / ssum], axis=1)

    oh1 = (lane == i1).astype(jnp.int32)
    oh2 = (lane == i2).astype(jnp.int32)
    cnt = oh1 + oh2
    c = cnt
    sh = 1
    while sh < T:
        c = c + jnp.concatenate(
            [jnp.zeros((sh, E), jnp.int32), c[: T - sh]], axis=0)
        sh *= 2
    counts = c[T - 1: T, :]
    cex = c - cnt
    pc = ((counts + BT - 1) // BT) * BT
    ic = pc
    for sh in (1, 2, 4):
        ic = ic + jnp.concatenate(
            [jnp.zeros((1, sh), jnp.int32), ic[:, : E - sh]], axis=1)
    off = ic - pc

    pos0 = jnp.sum(oh1 * (off + cex), axis=1, keepdims=True)
    pos1 = jnp.sum(oh2 * (off + cex), axis=1, keepdims=True)
    pos_ref[...] = jnp.concatenate([pos0, pos1], axis=1)

    blk = lax.broadcasted_iota(jnp.int32, (1, NBPAD), 1)
    emap = jnp.zeros((1, NBPAD), jnp.int32)
    valid = jnp.zeros((1, NBPAD), jnp.int32)
    for e in range(E):
        lo = off[0:1, e:e + 1] // BT
        hi = (off[0:1, e:e + 1] + pc[0:1, e:e + 1]) // BT
        ine = (blk >= lo) & (blk < hi)
        emap = jnp.where(ine, e, emap)
        valid = valid | ine.astype(jnp.int32)
    lane8 = lax.broadcasted_iota(jnp.int32, (1, E), 1)
    elast = jnp.max(jnp.where(pc > 0, lane8, 0))
    emap_ref[...] = jnp.where(valid == 1, emap, elast)
    evalid_ref[...] = jnp.where(valid == 1, emap, -1)


def _router_call(x, gate_w):
    return pl.pallas_call(
        _router_body,
        out_shape=[
            jax.ShapeDtypeStruct((T, TOPK), jnp.int32),
            jax.ShapeDtypeStruct((T, TOPK), jnp.float32),
            jax.ShapeDtypeStruct((1, NBPAD), jnp.int32),
            jax.ShapeDtypeStruct((1, NBPAD), jnp.int32),
            jax.ShapeDtypeStruct((T, H), jnp.float32),
        ],
    )(x, gate_w)


def _scatter_body(pos_hbm, w_hbm, st_hbm, wsc_hbm, pos_v, w_v, st_v, wsc_v):
    cid = lax.axis_index("c")
    sid = lax.axis_index("s")

    @pl.when((cid == 0) & (sid == 0))
    def _():
        pltpu.sync_copy(pos_hbm, pos_v)
        pltpu.sync_copy(w_hbm, w_v)

        iota16 = lax.iota(jnp.int32, 16)

        def zero_body(i, carry):
            st_v[pl.ds(i * 16, 16)] = (i * 16 + iota16) & (T - 1)
            wsc_v[pl.ds(i * 16, 16)] = jnp.zeros((16,), jnp.float32)
            return carry

        lax.fori_loop(0, P // 16, zero_body, 0)

        def sc_body(i, carry):
            idx = pos_v[pl.ds(i * 16, 16)]
            tok = (i * 16 + iota16) // 2
            plsc.store_scatter(st_v, [idx], tok)
            wv = w_v[pl.ds(i * 16, 16)]
            plsc.store_scatter(wsc_v, [idx], wv)
            return carry

        lax.fori_loop(0, S // 16, sc_body, 0)
        pltpu.sync_copy(st_v, st_hbm)
        pltpu.sync_copy(wsc_v, wsc_hbm)


_RW = P // NW
_CH = 80


def _gather_body(st_hbm, x_hbm, xs_hbm, ia_v, ib_v, ra_v, rb_v, sg, sw):
    wid = lax.axis_index("s") * 2 + lax.axis_index("c")
    base = wid * _RW
    pltpu.sync_copy(st_hbm.at[pl.ds(base, _CH)], ia_v)
    g0 = pltpu.async_copy(x_hbm.at[ia_v], ra_v, sg)
    pltpu.sync_copy(st_hbm.at[pl.ds(base + _CH, _CH)], ib_v)
    g1 = pltpu.async_copy(x_hbm.at[ib_v], rb_v, sg)
    g0.wait()
    w0 = pltpu.async_copy(ra_v, xs_hbm.at[pl.ds(base, _CH)], sw)
    g1.wait()
    w1 = pltpu.async_copy(rb_v, xs_hbm.at[pl.ds(base + _CH, _CH)], sw)
    w0.wait()
    w1.wait()


def _mm_body(emap_ref, evalid_ref, xs_ref, ws_ref, w2_ref, wsc_ref, ys_ref):
    b = pl.program_id(0)

    @pl.when(evalid_ref[b] >= 0)
    def _():
        xs = xs_ref[...]
        wfull = ws_ref[0]
        gu = lax.dot_general(xs, wfull, (((1,), (1,)), ((), ())),
                             preferred_element_type=jnp.float32)
        g = gu[:, :I]
        u = gu[:, I:]
        h = (g * (1.0 / (1.0 + jnp.exp(-g)))) * u
        w2 = w2_ref[0]
        part = lax.dot_general(h, w2, (((1,), (1,)), ((), ())),
                               preferred_element_type=jnp.float32)
        ys_ref[...] = part * wsc_ref[...]


def _mm_call(emap, evalid, xs, ws, w2s, wsc):
    grid_spec = pltpu.PrefetchScalarGridSpec(
        num_scalar_prefetch=2,
        grid=(NBLK,),
        in_specs=[
            pl.BlockSpec((BT, H), lambda b, em, ev: (b, 0)),
            pl.BlockSpec((1, 2 * I, H), lambda b, em, ev: (em[b], 0, 0)),
            pl.BlockSpec((1, H, I), lambda b, em, ev: (em[b], 0, 0)),
            pl.BlockSpec((BT, 1), lambda b, em, ev: (b, 0)),
        ],
        out_specs=pl.BlockSpec((BT, H), lambda b, em, ev: (b, 0)),
    )
    return pl.pallas_call(
        _mm_body,
        grid_spec=grid_spec,
        out_shape=jax.ShapeDtypeStruct((P, H), jnp.float32),
    )(emap, evalid, xs, ws, w2s, wsc)


_TW = T // NW


def _combine_body(pos0_hbm, pos1_hbm, ys_hbm, out_hbm, i0_v, i1_v, a_v, b_v,
                  sem):
    wid = lax.axis_index("s") * 2 + lax.axis_index("c")
    base = wid * _TW
    pltpu.sync_copy(pos0_hbm.at[pl.ds(base, _TW)], i0_v)
    pltpu.sync_copy(pos1_hbm.at[pl.ds(base, _TW)], i1_v)
    cp0 = pltpu.async_copy(ys_hbm.at[i0_v], a_v, sem)
    cp1 = pltpu.async_copy(ys_hbm.at[i1_v], b_v, sem)
    cp0.wait()
    cp1.wait()

    def add_body(r, carry):
        for j in range(H // 16):
            sl = pl.ds(j * 16, 16)
            a_v[r, sl] = a_v[r, sl] + b_v[r, sl]
        return carry

    lax.fori_loop(0, _TW, add_body, 0)
    pltpu.sync_copy(a_v, out_hbm.at[pl.ds(base, _TW)])


@functools.lru_cache(maxsize=1)
def _sc_kernels():
    mesh = plsc.VectorSubcoreMesh(core_axis_name="c", subcore_axis_name="s")
    params = pltpu.CompilerParams(needs_layout_passes=False)
    scatter = pl.kernel(
        _scatter_body, mesh=mesh, compiler_params=params,
        out_type=[jax.ShapeDtypeStruct((P,), jnp.int32),
                  jax.ShapeDtypeStruct((P,), jnp.float32)],
        scratch_types=[pltpu.VMEM((S,), jnp.int32),
                       pltpu.VMEM((S,), jnp.float32),
                       pltpu.VMEM((P,), jnp.int32),
                       pltpu.VMEM((P,), jnp.float32)],
    )
    gather = pl.kernel(
        _gather_body, mesh=mesh, compiler_params=params,
        out_type=jax.ShapeDtypeStruct((P, H), jnp.float32),
        scratch_types=[pltpu.VMEM((_CH,), jnp.int32),
                       pltpu.VMEM((_CH,), jnp.int32),
                       pltpu.VMEM((_CH, H), jnp.float32),
                       pltpu.VMEM((_CH, H), jnp.float32),
                       pltpu.SemaphoreType.DMA,
                       pltpu.SemaphoreType.DMA],
    )
    combine = pl.kernel(
        _combine_body, mesh=mesh, compiler_params=params,
        out_type=jax.ShapeDtypeStruct((T, H), jnp.float32),
        scratch_types=[pltpu.VMEM((_TW,), jnp.int32),
                       pltpu.VMEM((_TW,), jnp.int32),
                       pltpu.VMEM((_TW, H), jnp.float32),
                       pltpu.VMEM((_TW, H), jnp.float32),
                       pltpu.SemaphoreType.DMA],
    )
    return scatter, gather, combine


def kernel(hidden_states, gate_w, ws, w2s):
    scatter, gather, combine = _sc_kernels()
    x = hidden_states.reshape(T, H)
    pos, wtk, emap2d, evalid2d, xcopy = _router_call(x, gate_w)
    emap = emap2d.reshape(NBPAD)
    evalid = evalid2d.reshape(NBPAD)
    st, wsc = scatter(pos.reshape(S), wtk.reshape(S))
    xs = gather(st, xcopy)
    ys = _mm_call(emap, evalid, xs, ws, w2s, wsc.reshape(P, 1))
    out = combine(pos[:, 0], pos[:, 1], ys)
    return out.reshape(T, H)


# speedup vs baseline: 1.2686x; 1.2686x over previous
import functools

import jax
import jax.numpy as jnp
from jax import lax
from jax.experimental import pallas as pl
from jax.experimental.pallas import tpu as pltpu
from jax.experimental.pallas import tpu_sc as plsc

E = 8
TOPK = 2
H = 768
I = 1536
T = 2048
S = T * TOPK
BT = 256
P = 6144
NBLK = P // BT
NBPAD = 128
NW = 32


def _router_body(x_ref, gw_ref, pos_ref, wtk_ref, emap_ref, evalid_ref,
                 xcopy_ref):
    x = x_ref[...]
    xcopy_ref[...] = x
    gw = gw_ref[...]
    logits = lax.dot_general(x, gw, (((1,), (1,)), ((), ())),
                             preferred_element_type=jnp.float32)
    m = jnp.max(logits, axis=1, keepdims=True)
    p = jnp.exp(logits - m)
    p = p / jnp.sum(p, axis=1, keepdims=True)

    lane = lax.broadcasted_iota(jnp.int32, (T, E), 1)
    m1 = jnp.max(p, axis=1, keepdims=True)
    i1 = jnp.min(jnp.where(p == m1, lane, E), axis=1, keepdims=True)
    p2 = jnp.where(lane == i1, -jnp.inf, p)
    m2 = jnp.max(p2, axis=1, keepdims=True)
    i2 = jnp.min(jnp.where(p2 == m2, lane, E), axis=1, keepdims=True)
    ssum = m1 + m2
    wtk_ref[...] = jnp.concatenate([m1 / ssum, m2 / ssum], axis=1)

    oh1 = (lane == i1).astype(jnp.int32)
    oh2 = (lane == i2).astype(jnp.int32)
    cnt = oh1 + oh2
    c = cnt
    sh = 1
    while sh < T:
        c = c + jnp.concatenate(
            [jnp.zeros((sh, E), jnp.int32), c[: T - sh]], axis=0)
        sh *= 2
    counts = c[T - 1: T, :]
    cex = c - cnt
    pc = ((counts + BT - 1) // BT) * BT
    ic = pc
    for sh in (1, 2, 4):
        ic = ic + jnp.concatenate(
            [jnp.zeros((1, sh), jnp.int32), ic[:, : E - sh]], axis=1)
    off = ic - pc

    pos0 = jnp.sum(oh1 * (off + cex), axis=1, keepdims=True)
    pos1 = jnp.sum(oh2 * (off + cex), axis=1, keepdims=True)
    pos_ref[...] = jnp.concatenate([pos0, pos1], axis=1)

    blk = lax.broadcasted_iota(jnp.int32, (1, NBPAD), 1)
    emap = jnp.zeros((1, NBPAD), jnp.int32)
    valid = jnp.zeros((1, NBPAD), jnp.int32)
    for e in range(E):
        lo = off[0:1, e:e + 1] // BT
        hi = (off[0:1, e:e + 1] + pc[0:1, e:e + 1]) // BT
        ine = (blk >= lo) & (blk < hi)
        emap = jnp.where(ine, e, emap)
        valid = valid | ine.astype(jnp.int32)
    lane8 = lax.broadcasted_iota(jnp.int32, (1, E), 1)
    elast = jnp.max(jnp.where(pc > 0, lane8, 0))
    emap_ref[...] = jnp.where(valid == 1, emap, elast)
    evalid_ref[...] = jnp.where(valid == 1, emap, -1)


def _router_call(x, gate_w):
    return pl.pallas_call(
        _router_body,
        out_shape=[
            jax.ShapeDtypeStruct((T, TOPK), jnp.int32),
            jax.ShapeDtypeStruct((T, TOPK), jnp.float32),
            jax.ShapeDtypeStruct((1, NBPAD), jnp.int32),
            jax.ShapeDtypeStruct((1, NBPAD), jnp.int32),
            jax.ShapeDtypeStruct((T, H), jnp.float32),
        ],
    )(x, gate_w)


_RW = P // NW
_CH = 64


def _dispatch_body(pos_hbm, w_hbm, x_hbm, xs_hbm, wsc_hbm,
                   pos_v, w_v, st_v, wsc_v, ia_v, ib_v, ra_v, rb_v,
                   st_sh, sg, sw):
    cid = lax.axis_index("c")
    sid = lax.axis_index("s")

    @pl.when(sid == 0)
    def _():
        pltpu.sync_copy(pos_hbm, pos_v)
        pltpu.sync_copy(w_hbm, w_v)

        iota16 = lax.iota(jnp.int32, 16)

        def zero_body(i, carry):
            st_v[pl.ds(i * 16, 16)] = (i * 16 + iota16) & (T - 1)
            wsc_v[pl.ds(i * 16, 16)] = jnp.zeros((16,), jnp.float32)
            return carry

        lax.fori_loop(0, P // 16, zero_body, 0)

        def sc_body(i, carry):
            idx = pos_v[pl.ds(i * 16, 16)]
            tok = (i * 16 + iota16) // 2
            plsc.store_scatter(st_v, [idx], tok)
            wv = w_v[pl.ds(i * 16, 16)]
            plsc.store_scatter(wsc_v, [idx], wv)
            return carry

        lax.fori_loop(0, S // 16, sc_body, 0)
        pltpu.sync_copy(st_v, st_sh)

        @pl.when(cid == 0)
        def _():
            pltpu.sync_copy(wsc_v, wsc_hbm)

    plsc.subcore_barrier()

    wid = sid * 2 + cid
    base = wid * _RW
    pltpu.sync_copy(st_sh.at[pl.ds(base, _CH)], ia_v)
    g0 = pltpu.async_copy(x_hbm.at[ia_v], ra_v, sg)
    pltpu.sync_copy(st_sh.at[pl.ds(base + _CH, _CH)], ib_v)
    g1 = pltpu.async_copy(x_hbm.at[ib_v], rb_v, sg)
    g0.wait()
    w0 = pltpu.async_copy(ra_v, xs_hbm.at[pl.ds(base, _CH)], sw)
    g1.wait()
    w1 = pltpu.async_copy(rb_v, xs_hbm.at[pl.ds(base + _CH, _CH)], sw)
    w0.wait()
    pltpu.sync_copy(st_sh.at[pl.ds(base + 2 * _CH, _CH)], ia_v)
    g2 = pltpu.async_copy(x_hbm.at[ia_v], ra_v, sg)
    g2.wait()
    w2 = pltpu.async_copy(ra_v, xs_hbm.at[pl.ds(base + 2 * _CH, _CH)], sw)
    w1.wait()
    w2.wait()


def _mm_body(emap_ref, evalid_ref, xs_ref, ws_ref, w2_ref, wsc_ref, ys_ref):
    b = pl.program_id(0)

    @pl.when(evalid_ref[b] >= 0)
    def _():
        xs = xs_ref[...]
        wfull = ws_ref[0]
        gu = lax.dot_general(xs, wfull, (((1,), (1,)), ((), ())),
                             preferred_element_type=jnp.float32)
        g = gu[:, :I]
        u = gu[:, I:]
        h = (g * (1.0 / (1.0 + jnp.exp(-g)))) * u
        w2 = w2_ref[0]
        part = lax.dot_general(h, w2, (((1,), (1,)), ((), ())),
                               preferred_element_type=jnp.float32)
        ys_ref[...] = part * wsc_ref[...]


def _mm_call(emap, evalid, xs, ws, w2s, wsc):
    grid_spec = pltpu.PrefetchScalarGridSpec(
        num_scalar_prefetch=2,
        grid=(NBLK,),
        in_specs=[
            pl.BlockSpec((BT, H), lambda b, em, ev: (b, 0)),
            pl.BlockSpec((1, 2 * I, H), lambda b, em, ev: (em[b], 0, 0)),
            pl.BlockSpec((1, H, I), lambda b, em, ev: (em[b], 0, 0)),
            pl.BlockSpec((BT, 1), lambda b, em, ev: (b, 0)),
        ],
        out_specs=pl.BlockSpec((BT, H), lambda b, em, ev: (b, 0)),
    )
    return pl.pallas_call(
        _mm_body,
        grid_spec=grid_spec,
        out_shape=jax.ShapeDtypeStruct((P, H), jnp.float32),
    )(emap, evalid, xs, ws, w2s, wsc)


_TW = T // NW


def _combine_body(pos0_hbm, pos1_hbm, ys_hbm, out_hbm, i0_v, i1_v, a_v, b_v,
                  sem):
    wid = lax.axis_index("s") * 2 + lax.axis_index("c")
    base = wid * _TW
    pltpu.sync_copy(pos0_hbm.at[pl.ds(base, _TW)], i0_v)
    pltpu.sync_copy(pos1_hbm.at[pl.ds(base, _TW)], i1_v)
    cp0 = pltpu.async_copy(ys_hbm.at[i0_v], a_v, sem)
    cp1 = pltpu.async_copy(ys_hbm.at[i1_v], b_v, sem)
    cp0.wait()
    cp1.wait()

    def add_body(r, carry):
        for j in range(H // 16):
            sl = pl.ds(j * 16, 16)
            a_v[r, sl] = a_v[r, sl] + b_v[r, sl]
        return carry

    lax.fori_loop(0, _TW, add_body, 0)
    pltpu.sync_copy(a_v, out_hbm.at[pl.ds(base, _TW)])


@functools.lru_cache(maxsize=1)
def _sc_kernels():
    mesh = plsc.VectorSubcoreMesh(core_axis_name="c", subcore_axis_name="s")
    params = pltpu.CompilerParams(needs_layout_passes=False)
    dispatch = pl.kernel(
        _dispatch_body, mesh=mesh, compiler_params=params,
        out_type=[jax.ShapeDtypeStruct((P, H), jnp.float32),
                  jax.ShapeDtypeStruct((P,), jnp.float32)],
        scratch_types=[pltpu.VMEM((S,), jnp.int32),
                       pltpu.VMEM((S,), jnp.float32),
                       pltpu.VMEM((P,), jnp.int32),
                       pltpu.VMEM((P,), jnp.float32),
                       pltpu.VMEM((_CH,), jnp.int32),
                       pltpu.VMEM((_CH,), jnp.int32),
                       pltpu.VMEM((_CH, H), jnp.float32),
                       pltpu.VMEM((_CH, H), jnp.float32),
                       pltpu.VMEM_SHARED((P,), jnp.int32),
                       pltpu.SemaphoreType.DMA,
                       pltpu.SemaphoreType.DMA],
    )
    combine = pl.kernel(
        _combine_body, mesh=mesh, compiler_params=params,
        out_type=jax.ShapeDtypeStruct((T, H), jnp.float32),
        scratch_types=[pltpu.VMEM((_TW,), jnp.int32),
                       pltpu.VMEM((_TW,), jnp.int32),
                       pltpu.VMEM((_TW, H), jnp.float32),
                       pltpu.VMEM((_TW, H), jnp.float32),
                       pltpu.SemaphoreType.DMA],
    )
    return dispatch, combine


def kernel(hidden_states, gate_w, ws, w2s):
    dispatch, combine = _sc_kernels()
    x = hidden_states.reshape(T, H)
    pos, wtk, emap2d, evalid2d, xcopy = _router_call(x, gate_w)
    emap = emap2d.reshape(NBPAD)
    evalid = evalid2d.reshape(NBPAD)
    xs, wsc = dispatch(pos.reshape(S), wtk.reshape(S), xcopy)
    ys = _mm_call(emap, evalid, xs, ws, w2s, wsc.reshape(P, 1))
    out = combine(pos[:, 0], pos[:, 1], ys)
    return out.reshape(T, H)


# speedup vs baseline: 1.2870x; 1.0145x over previous
import functools

import jax
import jax.numpy as jnp
from jax import lax
from jax.experimental import pallas as pl
from jax.experimental.pallas import tpu as pltpu
from jax.experimental.pallas import tpu_sc as plsc

E = 8
TOPK = 2
H = 768
I = 1536
T = 2048
S = T * TOPK
BT = 256
P = 6144
NBLK = P // BT
NBPAD = 128
NW = 32


def _router_body(x_ref, gw_ref, pos_ref, wtk_ref, emap_ref, evalid_ref):
    x = x_ref[...]
    gw = gw_ref[...]
    logits = lax.dot_general(x, gw, (((1,), (1,)), ((), ())),
                             preferred_element_type=jnp.float32)
    m = jnp.max(logits, axis=1, keepdims=True)
    p = jnp.exp(logits - m)
    p = p / jnp.sum(p, axis=1, keepdims=True)

    lane = lax.broadcasted_iota(jnp.int32, (T, E), 1)
    m1 = jnp.max(p, axis=1, keepdims=True)
    i1 = jnp.min(jnp.where(p == m1, lane, E), axis=1, keepdims=True)
    p2 = jnp.where(lane == i1, -jnp.inf, p)
    m2 = jnp.max(p2, axis=1, keepdims=True)
    i2 = jnp.min(jnp.where(p2 == m2, lane, E), axis=1, keepdims=True)
    ssum = m1 + m2
    wtk_ref[...] = jnp.concatenate([m1 / ssum, m2 / ssum], axis=1)

    oh1 = (lane == i1).astype(jnp.int32)
    oh2 = (lane == i2).astype(jnp.int32)
    cnt = oh1 + oh2
    c = cnt
    sh = 1
    while sh < T:
        c = c + jnp.concatenate(
            [jnp.zeros((sh, E), jnp.int32), c[: T - sh]], axis=0)
        sh *= 2
    counts = c[T - 1: T, :]
    cex = c - cnt
    pc = ((counts + BT - 1) // BT) * BT
    ic = pc
    for sh in (1, 2, 4):
        ic = ic + jnp.concatenate(
            [jnp.zeros((1, sh), jnp.int32), ic[:, : E - sh]], axis=1)
    off = ic - pc

    pos0 = jnp.sum(oh1 * (off + cex), axis=1, keepdims=True)
    pos1 = jnp.sum(oh2 * (off + cex), axis=1, keepdims=True)
    pos_ref[...] = jnp.concatenate([pos0, pos1], axis=1)

    blk = lax.broadcasted_iota(jnp.int32, (1, NBPAD), 1)
    emap = jnp.zeros((1, NBPAD), jnp.int32)
    valid = jnp.zeros((1, NBPAD), jnp.int32)
    for e in range(E):
        lo = off[0:1, e:e + 1] // BT
        hi = (off[0:1, e:e + 1] + pc[0:1, e:e + 1]) // BT
        ine = (blk >= lo) & (blk < hi)
        emap = jnp.where(ine, e, emap)
        valid = valid | ine.astype(jnp.int32)
    lane8 = lax.broadcasted_iota(jnp.int32, (1, E), 1)
    elast = jnp.max(jnp.where(pc > 0, lane8, 0))
    emap_ref[...] = jnp.where(valid == 1, emap, elast)
    evalid_ref[...] = jnp.where(valid == 1, emap, -1)


def _router_call(x, gate_w):
    return pl.pallas_call(
        _router_body,
        out_shape=[
            jax.ShapeDtypeStruct((T, TOPK), jnp.int32),
            jax.ShapeDtypeStruct((T, TOPK), jnp.float32),
            jax.ShapeDtypeStruct((1, NBPAD), jnp.int32),
            jax.ShapeDtypeStruct((1, NBPAD), jnp.int32),
        ],
    )(x, gate_w)


_RW = P // NW
_CH = 64


def _dispatch_body(pos_hbm, w_hbm, x_hbm, xs_hbm, wsc_hbm,
                   pos_v, w_v, st_v, wsc_v, ia_v, ib_v, ra_v, rb_v,
                   st_sh, sg, sw):
    cid = lax.axis_index("c")
    sid = lax.axis_index("s")

    @pl.when(sid == 0)
    def _():
        pltpu.sync_copy(pos_hbm, pos_v)
        pltpu.sync_copy(w_hbm, w_v)

        iota16 = lax.iota(jnp.int32, 16)

        def zero_body(i, carry):
            st_v[pl.ds(i * 16, 16)] = (i * 16 + iota16) & (T - 1)
            wsc_v[pl.ds(i * 16, 16)] = jnp.zeros((16,), jnp.float32)
            return carry

        lax.fori_loop(0, P // 16, zero_body, 0)

        def sc_body(i, carry):
            idx = pos_v[pl.ds(i * 16, 16)]
            tok = (i * 16 + iota16) // 2
            plsc.store_scatter(st_v, [idx], tok)
            wv = w_v[pl.ds(i * 16, 16)]
            plsc.store_scatter(wsc_v, [idx], wv)
            return carry

        lax.fori_loop(0, S // 16, sc_body, 0)
        pltpu.sync_copy(st_v, st_sh)

        @pl.when(cid == 0)
        def _():
            pltpu.sync_copy(wsc_v, wsc_hbm)

    plsc.subcore_barrier()

    wid = sid * 2 + cid
    base = wid * _RW
    pltpu.sync_copy(st_sh.at[pl.ds(base, _CH)], ia_v)
    g0 = pltpu.async_copy(x_hbm.at[ia_v], ra_v, sg)
    pltpu.sync_copy(st_sh.at[pl.ds(base + _CH, _CH)], ib_v)
    g1 = pltpu.async_copy(x_hbm.at[ib_v], rb_v, sg)
    g0.wait()
    w0 = pltpu.async_copy(ra_v, xs_hbm.at[pl.ds(base, _CH)], sw)
    g1.wait()
    w1 = pltpu.async_copy(rb_v, xs_hbm.at[pl.ds(base + _CH, _CH)], sw)
    w0.wait()
    pltpu.sync_copy(st_sh.at[pl.ds(base + 2 * _CH, _CH)], ia_v)
    g2 = pltpu.async_copy(x_hbm.at[ia_v], ra_v, sg)
    g2.wait()
    w2 = pltpu.async_copy(ra_v, xs_hbm.at[pl.ds(base + 2 * _CH, _CH)], sw)
    w1.wait()
    w2.wait()


def _mm_body(emap_ref, evalid_ref, xs_ref, ws_ref, w2_ref, wsc_ref, ys_ref):
    b = pl.program_id(0)

    @pl.when(evalid_ref[b] >= 0)
    def _():
        xs = xs_ref[...]
        wfull = ws_ref[0]
        gu = lax.dot_general(xs, wfull, (((1,), (1,)), ((), ())),
                             preferred_element_type=jnp.float32)
        g = gu[:, :I]
        u = gu[:, I:]
        h = (g * (1.0 / (1.0 + jnp.exp(-g)))) * u
        w2 = w2_ref[0]
        part = lax.dot_general(h, w2, (((1,), (1,)), ((), ())),
                               preferred_element_type=jnp.float32)
        ys_ref[...] = part * wsc_ref[...]


def _mm_call(emap, evalid, xs, ws, w2s, wsc):
    grid_spec = pltpu.PrefetchScalarGridSpec(
        num_scalar_prefetch=2,
        grid=(NBLK,),
        in_specs=[
            pl.BlockSpec((BT, H), lambda b, em, ev: (b, 0)),
            pl.BlockSpec((1, 2 * I, H), lambda b, em, ev: (em[b], 0, 0)),
            pl.BlockSpec((1, H, I), lambda b, em, ev: (em[b], 0, 0)),
            pl.BlockSpec((BT, 1), lambda b, em, ev: (b, 0)),
        ],
        out_specs=pl.BlockSpec((BT, H), lambda b, em, ev: (b, 0)),
    )
    return pl.pallas_call(
        _mm_body,
        grid_spec=grid_spec,
        out_shape=jax.ShapeDtypeStruct((P, H), jnp.float32),
    )(emap, evalid, xs, ws, w2s, wsc)


_TW = T // NW


def _combine_body(pos0_hbm, pos1_hbm, ys_hbm, out_hbm, i0_v, i1_v, a_v, b_v,
                  sem):
    wid = lax.axis_index("s") * 2 + lax.axis_index("c")
    base = wid * _TW
    pltpu.sync_copy(pos0_hbm.at[pl.ds(base, _TW)], i0_v)
    pltpu.sync_copy(pos1_hbm.at[pl.ds(base, _TW)], i1_v)
    cp0 = pltpu.async_copy(ys_hbm.at[i0_v], a_v, sem)
    cp1 = pltpu.async_copy(ys_hbm.at[i1_v], b_v, sem)
    cp0.wait()
    cp1.wait()

    def add_body(r, carry):
        for j in range(H // 16):
            sl = pl.ds(j * 16, 16)
            a_v[r, sl] = a_v[r, sl] + b_v[r, sl]
        return carry

    lax.fori_loop(0, _TW, add_body, 0)
    pltpu.sync_copy(a_v, out_hbm.at[pl.ds(base, _TW)])


@functools.lru_cache(maxsize=1)
def _sc_kernels():
    mesh = plsc.VectorSubcoreMesh(core_axis_name="c", subcore_axis_name="s")
    params = pltpu.CompilerParams(needs_layout_passes=False)
    dispatch = pl.kernel(
        _dispatch_body, mesh=mesh, compiler_params=params,
        out_type=[jax.ShapeDtypeStruct((P, H), jnp.float32),
                  jax.ShapeDtypeStruct((P,), jnp.float32)],
        scratch_types=[pltpu.VMEM((S,), jnp.int32),
                       pltpu.VMEM((S,), jnp.float32),
                       pltpu.VMEM((P,), jnp.int32),
                       pltpu.VMEM((P,), jnp.float32),
                       pltpu.VMEM((_CH,), jnp.int32),
                       pltpu.VMEM((_CH,), jnp.int32),
                       pltpu.VMEM((_CH, H), jnp.float32),
                       pltpu.VMEM((_CH, H), jnp.float32),
                       pltpu.VMEM_SHARED((P,), jnp.int32),
                       pltpu.SemaphoreType.DMA,
                       pltpu.SemaphoreType.DMA],
    )
    combine = pl.kernel(
        _combine_body, mesh=mesh, compiler_params=params,
        out_type=jax.ShapeDtypeStruct((T, H), jnp.float32),
        scratch_types=[pltpu.VMEM((_TW,), jnp.int32),
                       pltpu.VMEM((_TW,), jnp.int32),
                       pltpu.VMEM((_TW, H), jnp.float32),
                       pltpu.VMEM((_TW, H), jnp.float32),
                       pltpu.SemaphoreType.DMA],
    )
    return dispatch, combine


def kernel(hidden_states, gate_w, ws, w2s):
    dispatch, combine = _sc_kernels()
    x = hidden_states.reshape(T, H)
    pos, wtk, emap2d, evalid2d = _router_call(x, gate_w)
    emap = emap2d.reshape(NBPAD)
    evalid = evalid2d.reshape(NBPAD)
    xs, wsc = dispatch(pos.reshape(S), wtk.reshape(S), x)
    ys = _mm_call(emap, evalid, xs, ws, w2s, wsc.reshape(P, 1))
    out = combine(pos[:, 0], pos[:, 1], ys)
    return out.reshape(T, H)
